# baseline (speedup 1.0000x reference)
import functools

import jax
import jax.numpy as jnp
from jax import lax
from jax.experimental import pallas as pl
from jax.experimental.pallas import tpu as pltpu
from jax.experimental.pallas import tpu_sc as plsc

_NC = 2
_NS = 16
_NW = _NC * _NS
_G = 64


def _make_segsum(n, e, d):
    epw = e // _NW
    c = 125
    nchunk = epw // c
    r0 = (n // _NS) // 8 * 8
    rlast = n - (_NS - 1) * r0
    mesh = plsc.VectorSubcoreMesh(core_axis_name="c", subcore_axis_name="s")

    @functools.partial(
        pl.kernel,
        out_type=jax.ShapeDtypeStruct((_NC, n, d), jnp.float32),
        mesh=mesh,
        compiler_params=pltpu.CompilerParams(use_tc_tiling_on_sc=False),
        scratch_types=[
            pltpu.VMEM((nchunk, c), jnp.int32),
            pltpu.VMEM((nchunk, c), jnp.int32),
            pltpu.VMEM((c, d), jnp.float32),
            pltpu.VMEM((c, d), jnp.float32),
            pltpu.VMEM_SHARED((n, d), jnp.float32),
            pltpu.SemaphoreType.DMA,
            pltpu.SemaphoreType.DMA,
            pltpu.SemaphoreType.DMA,
        ],
    )
    def segsum(src_hbm, dst_hbm, y_hbm, zeros_hbm, out_hbm,
               idx_s, idx_d, rows0, rows1, agg_sh, sem0, sem1, sem_i):
        cid = lax.axis_index("c")
        sid = lax.axis_index("s")
        wid = sid * _NC + cid

        @pl.when(sid < _NS - 1)
        def _():
            pltpu.sync_copy(zeros_hbm.at[pl.ds(0, r0)],
                            agg_sh.at[pl.ds(sid * r0, r0)])

        @pl.when(sid == _NS - 1)
        def _():
            pltpu.sync_copy(zeros_hbm,
                            agg_sh.at[pl.ds((_NS - 1) * r0, rlast)])

        pltpu.async_copy(src_hbm.at[wid], idx_s, sem_i)
        pltpu.async_copy(dst_hbm.at[wid], idx_d, sem_i).wait()
        pltpu.make_async_copy(src_hbm.at[wid], idx_s, sem_i).wait()
        plsc.subcore_barrier()

        pltpu.async_copy(y_hbm.at[idx_s.at[0]], rows0, sem0)

        def body(k, carry):
            ch0 = 2 * k
            ch1 = ch0 + 1
            pltpu.async_copy(y_hbm.at[idx_s.at[ch1]], rows1, sem1)
            pltpu.make_async_copy(y_hbm.at[idx_s.at[ch0]], rows0, sem0).wait()
            pltpu.sync_copy(rows0, agg_sh.at[idx_d.at[ch0]], add=True)

            @pl.when(ch0 + 2 < nchunk)
            def _():
                pltpu.async_copy(y_hbm.at[idx_s.at[ch0 + 2]], rows0, sem0)

            pltpu.make_async_copy(y_hbm.at[idx_s.at[ch1]], rows1, sem1).wait()
            pltpu.sync_copy(rows1, agg_sh.at[idx_d.at[ch1]], add=True)
            return carry

        lax.fori_loop(0, nchunk // 2, body, 0, unroll=False)
        plsc.subcore_barrier()

        @pl.when(sid < _NS - 1)
        def _():
            pltpu.sync_copy(agg_sh.at[pl.ds(sid * r0, r0)],
                            out_hbm.at[cid, pl.ds(sid * r0, r0)])

        @pl.when(sid == _NS - 1)
        def _():
            pltpu.sync_copy(agg_sh.at[pl.ds((_NS - 1) * r0, rlast)],
                            out_hbm.at[cid, pl.ds((_NS - 1) * r0, rlast)])

    return segsum


def _tc1_body(x_ref, w_ref, b_ref, am_ref):
    z = jnp.dot(x_ref[...], w_ref[...], preferred_element_type=jnp.float32)
    z = z + b_ref[...]
    a = jnp.maximum(z[:, :64], 0.0)
    m = jnp.maximum(z[:, 64:80], 0.0) * jnp.maximum(z[:, 80:96], 0.0)
    am_ref[...] = jnp.concatenate([a, m], axis=1)


def _tc2_body(am_ref, agga_ref, aggb_ref, wc_ref, bc_ref, h1a_ref, h1b_ref):
    agg = jnp.concatenate(
        [agga_ref[0] + agga_ref[1], aggb_ref[0] + aggb_ref[1]], axis=1)
    c = jnp.maximum(
        jnp.dot(agg, wc_ref[...], preferred_element_type=jnp.float32)
        + bc_ref[...], 0.0)
    h1a_ref[...] = am_ref[:, :64]
    h1b_ref[...] = jnp.concatenate([c, am_ref[:, 64:80]], axis=1)


def _tc3_body(nb, h1a_ref, h1b_ref, agga_ref, aggb_ref, w_ref, b_ref,
              wc_ref, bc_ref, batch_ref, wfc_ref, bfc_ref,
              out_ref, sum_ref, mx_ref, cnt_ref):
    i = pl.program_id(0)

    @pl.when(i == 0)
    def _():
        sum_ref[...] = jnp.zeros_like(sum_ref)
        mx_ref[...] = jnp.zeros_like(mx_ref)
        cnt_ref[...] = jnp.zeros_like(cnt_ref)

    agg = jnp.concatenate(
        [agga_ref[0] + agga_ref[1], aggb_ref[0] + aggb_ref[1]], axis=1)
    c2 = jnp.maximum(
        jnp.dot(agg, wc_ref[...], preferred_element_type=jnp.float32)
        + bc_ref[...], 0.0)
    h1 = jnp.concatenate([h1a_ref[...], h1b_ref[...]], axis=1)
    z = jnp.dot(h1, w_ref[...],
                preferred_element_type=jnp.float32) + b_ref[...]
    a2 = jnp.maximum(z[:, :64], 0.0)
    m2 = jnp.maximum(z[:, 64:80], 0.0) * jnp.maximum(z[:, 80:96], 0.0)
    h2 = jnp.concatenate([a2, c2, m2], axis=1)

    bsz = h2.shape[0]
    bid = batch_ref[0, 0, :].reshape(bsz, 1)
    g_lo = batch_ref[0, 0, 0]
    g_hi = batch_ref[0, 0, bsz - 1]

    def seg(g, carry):
        msk = (bid == g).astype(jnp.float32)
        mh = h2 * msk
        s = jnp.sum(mh, axis=0, keepdims=True)
        mx = jnp.max(mh, axis=0, keepdims=True)
        cnt = jnp.sum(msk)
        sum_ref[pl.ds(g, 1), :] += s
        mx_ref[pl.ds(g, 1), :] = jnp.maximum(mx_ref[pl.ds(g, 1), :], mx)
        cnt_ref[pl.ds(g, 1), :] += cnt
        return carry

    lax.fori_loop(g_lo, g_hi + 1, seg, 0)

    @pl.when(i == nb - 1)
    def _():
        mean = sum_ref[...] / jnp.maximum(cnt_ref[...], 1.0)
        pooled = jnp.concatenate([mean, mx_ref[...]], axis=1)
        logits = jnp.dot(pooled, wfc_ref[...],
                         preferred_element_type=jnp.float32) + bfc_ref[...]
        m = jnp.max(logits, axis=1, keepdims=True)
        lse = m + jnp.log(jnp.sum(jnp.exp(logits - m), axis=1, keepdims=True))
        out_ref[...] = logits - lse


def kernel(x, edge_index, batch, Wc1, bc1, W11, b11, W12, b12, W13, b13,
           Wc2, bc2, W21, b21, W22, b22, W23, b23, Wfc2, bfc2):
    n, dx = x.shape
    e = edge_index.shape[1]
    epw = e // _NW
    c = 125
    nchunk = epw // c
    rlast = n - (_NS - 1) * ((n // _NS) // 8 * 8)

    src3 = edge_index[0].reshape(_NW, nchunk, c)
    dst3 = edge_index[1].reshape(_NW, nchunk, c)
    zeros64 = jnp.zeros((rlast, 64), jnp.float32)
    zeros80 = jnp.zeros((rlast, 80), jnp.float32)

    Wam1 = jnp.concatenate([W11, W12, W13], axis=1)
    bam1 = jnp.concatenate([b11, b12, b13])[None, :]
    Wam2 = jnp.concatenate([W21, W22, W23], axis=1)
    bam2 = jnp.concatenate([b21, b22, b23])[None, :]

    bsz = 1000
    nb = n // bsz
    full = lambda shape: pl.BlockSpec(shape, lambda i: tuple(0 for _ in shape))

    am1 = pl.pallas_call(
        _tc1_body,
        grid=(nb,),
        in_specs=[
            pl.BlockSpec((bsz, dx), lambda i: (i, 0)),
            full((dx, 96)),
            full((1, 96)),
        ],
        out_specs=pl.BlockSpec((bsz, 80), lambda i: (i, 0)),
        out_shape=jax.ShapeDtypeStruct((n, 80), jnp.float32),
    )(x, Wam1, bam1)

    segsum64 = _make_segsum(n, e, 64)
    xa = x[:, :64]
    xb = x[:, 64:dx]
    aggxa = segsum64(src3, dst3, xa, zeros64)
    aggxb = segsum64(src3, dst3, xb, zeros64)

    h1a, h1b = pl.pallas_call(
        _tc2_body,
        grid=(nb,),
        in_specs=[
            pl.BlockSpec((bsz, 80), lambda i: (i, 0)),
            pl.BlockSpec((_NC, bsz, 64), lambda i: (0, i, 0)),
            pl.BlockSpec((_NC, bsz, 64), lambda i: (0, i, 0)),
            full((dx, 64)),
            full((1, 64)),
        ],
        out_specs=[
            pl.BlockSpec((bsz, 64), lambda i: (i, 0)),
            pl.BlockSpec((bsz, 80), lambda i: (i, 0)),
        ],
        out_shape=[
            jax.ShapeDtypeStruct((n, 64), jnp.float32),
            jax.ShapeDtypeStruct((n, 80), jnp.float32),
        ],
    )(am1, aggxa, aggxb, Wc1, bc1[None, :])

    segsum80 = _make_segsum(n, e, 80)
    aggha = segsum64(src3, dst3, h1a, zeros64)
    agghb = segsum80(src3, dst3, h1b, zeros80)

    batch3 = batch.reshape(nb, 1, bsz)
    out = pl.pallas_call(
        functools.partial(_tc3_body, nb),
        grid=(nb,),
        in_specs=[
            pl.BlockSpec((bsz, 64), lambda i: (i, 0)),
            pl.BlockSpec((bsz, 80), lambda i: (i, 0)),
            pl.BlockSpec((_NC, bsz, 64), lambda i: (0, i, 0)),
            pl.BlockSpec((_NC, bsz, 80), lambda i: (0, i, 0)),
            full((144, 96)),
            full((1, 96)),
            full((144, 64)),
            full((1, 64)),
            pl.BlockSpec((1, 1, bsz), lambda i: (i, 0, 0)),
            full((288, 2)),
            full((1, 2)),
        ],
        out_specs=pl.BlockSpec((_G, 2), lambda i: (0, 0)),
        out_shape=jax.ShapeDtypeStruct((_G, 2), jnp.float32),
        scratch_shapes=[
            pltpu.VMEM((_G, 144), jnp.float32),
            pltpu.VMEM((_G, 144), jnp.float32),
            pltpu.VMEM((_G, 144), jnp.float32),
        ],
    )(h1a, h1b, aggha, agghb, Wam2, bam2, Wc2, bc2[None, :], batch3,
      Wfc2, bfc2[None, :])
    return out


# speedup vs baseline: 1.0621x; 1.0621x over previous
import functools

import jax
import jax.numpy as jnp
from jax import lax
from jax.experimental import pallas as pl
from jax.experimental.pallas import tpu as pltpu
from jax.experimental.pallas import tpu_sc as plsc

_NC = 2
_NS = 16
_NW = _NC * _NS
_G = 64


def _make_segsum(n, e, d):
    epw = e // _NW
    c = 125
    nchunk = epw // c
    r0 = (n // _NS) // 8 * 8
    rlast = n - (_NS - 1) * r0
    mesh = plsc.VectorSubcoreMesh(core_axis_name="c", subcore_axis_name="s")

    @functools.partial(
        pl.kernel,
        out_type=jax.ShapeDtypeStruct((_NC, n, d), jnp.float32),
        mesh=mesh,
        compiler_params=pltpu.CompilerParams(use_tc_tiling_on_sc=False),
        scratch_types=[
            pltpu.VMEM((nchunk, c), jnp.int32),
            pltpu.VMEM((nchunk, c), jnp.int32),
            [pltpu.VMEM((c, d), jnp.float32)] * 4,
            pltpu.VMEM_SHARED((n, d), jnp.float32),
            [pltpu.SemaphoreType.DMA] * 4,
            [pltpu.SemaphoreType.DMA] * 4,
            pltpu.SemaphoreType.DMA,
        ],
    )
    def segsum(src_hbm, dst_hbm, y_hbm, zeros_hbm, out_hbm,
               idx_s, idx_d, rows, agg_sh, sem_g, sem_s, sem_i):
        cid = lax.axis_index("c")
        sid = lax.axis_index("s")
        wid = sid * _NC + cid

        @pl.when(sid < _NS - 1)
        def _():
            pltpu.sync_copy(zeros_hbm.at[pl.ds(0, r0)],
                            agg_sh.at[pl.ds(sid * r0, r0)])

        @pl.when(sid == _NS - 1)
        def _():
            pltpu.sync_copy(zeros_hbm,
                            agg_sh.at[pl.ds((_NS - 1) * r0, rlast)])

        pltpu.async_copy(src_hbm.at[wid], idx_s, sem_i)
        pltpu.async_copy(dst_hbm.at[wid], idx_d, sem_i).wait()
        pltpu.make_async_copy(src_hbm.at[wid], idx_s, sem_i).wait()
        plsc.subcore_barrier()

        nbuf = 4
        for j in range(nbuf):
            pltpu.async_copy(y_hbm.at[idx_s.at[j]], rows[j], sem_g[j])

        def body(k, carry):
            base = nbuf * k
            for j in range(nbuf):
                ch = base + j
                pltpu.make_async_copy(
                    y_hbm.at[idx_s.at[ch]], rows[j], sem_g[j]).wait()
                pltpu.async_copy(
                    rows[j], agg_sh.at[idx_d.at[ch]], sem_s[j], add=True)
            for j in range(nbuf):
                ch = base + j
                pltpu.make_async_copy(
                    rows[j], agg_sh.at[idx_d.at[ch]], sem_s[j]).wait()

                def refill(jj=j, nch=base + nbuf + j):
                    pltpu.async_copy(
                        y_hbm.at[idx_s.at[nch]], rows[jj], sem_g[jj])

                pl.when(base + nbuf + j < nchunk)(refill)
            return carry

        lax.fori_loop(0, nchunk // nbuf, body, 0, unroll=False)
        plsc.subcore_barrier()

        @pl.when(sid < _NS - 1)
        def _():
            pltpu.sync_copy(agg_sh.at[pl.ds(sid * r0, r0)],
                            out_hbm.at[cid, pl.ds(sid * r0, r0)])

        @pl.when(sid == _NS - 1)
        def _():
            pltpu.sync_copy(agg_sh.at[pl.ds((_NS - 1) * r0, rlast)],
                            out_hbm.at[cid, pl.ds((_NS - 1) * r0, rlast)])

    return segsum


def _tc1_body(x_ref, w_ref, b_ref, am_ref):
    z = jnp.dot(x_ref[...], w_ref[...], preferred_element_type=jnp.float32)
    z = z + b_ref[...]
    a = jnp.maximum(z[:, :64], 0.0)
    m = jnp.maximum(z[:, 64:80], 0.0) * jnp.maximum(z[:, 80:96], 0.0)
    am_ref[...] = jnp.concatenate([a, m], axis=1)


def _tc2_body(am_ref, agga_ref, aggb_ref, wc_ref, bc_ref, h1a_ref, h1b_ref):
    agg = jnp.concatenate(
        [agga_ref[0] + agga_ref[1], aggb_ref[0] + aggb_ref[1]], axis=1)
    c = jnp.maximum(
        jnp.dot(agg, wc_ref[...], preferred_element_type=jnp.float32)
        + bc_ref[...], 0.0)
    h1a_ref[...] = am_ref[:, :64]
    h1b_ref[...] = jnp.concatenate([c, am_ref[:, 64:80]], axis=1)


def _tc3_body(nb, h1a_ref, h1b_ref, agga_ref, aggb_ref, w_ref, b_ref,
              wc_ref, bc_ref, batch_ref, wfc_ref, bfc_ref,
              out_ref, sum_ref, mx_ref, cnt_ref):
    i = pl.program_id(0)

    @pl.when(i == 0)
    def _():
        sum_ref[...] = jnp.zeros_like(sum_ref)
        mx_ref[...] = jnp.zeros_like(mx_ref)
        cnt_ref[...] = jnp.zeros_like(cnt_ref)

    agg = jnp.concatenate(
        [agga_ref[0] + agga_ref[1], aggb_ref[0] + aggb_ref[1]], axis=1)
    c2 = jnp.maximum(
        jnp.dot(agg, wc_ref[...], preferred_element_type=jnp.float32)
        + bc_ref[...], 0.0)
    h1 = jnp.concatenate([h1a_ref[...], h1b_ref[...]], axis=1)
    z = jnp.dot(h1, w_ref[...],
                preferred_element_type=jnp.float32) + b_ref[...]
    a2 = jnp.maximum(z[:, :64], 0.0)
    m2 = jnp.maximum(z[:, 64:80], 0.0) * jnp.maximum(z[:, 80:96], 0.0)
    h2 = jnp.concatenate([a2, c2, m2], axis=1)

    bsz = h2.shape[0]
    bid = batch_ref[0, 0, :].reshape(bsz, 1)
    g_lo = batch_ref[0, 0, 0]
    g_hi = batch_ref[0, 0, bsz - 1]

    def seg(g, carry):
        msk = (bid == g).astype(jnp.float32)
        mh = h2 * msk
        s = jnp.sum(mh, axis=0, keepdims=True)
        mx = jnp.max(mh, axis=0, keepdims=True)
        cnt = jnp.sum(msk)
        sum_ref[pl.ds(g, 1), :] += s
        mx_ref[pl.ds(g, 1), :] = jnp.maximum(mx_ref[pl.ds(g, 1), :], mx)
        cnt_ref[pl.ds(g, 1), :] += cnt
        return carry

    lax.fori_loop(g_lo, g_hi + 1, seg, 0)

    @pl.when(i == nb - 1)
    def _():
        mean = sum_ref[...] / jnp.maximum(cnt_ref[...], 1.0)
        pooled = jnp.concatenate([mean, mx_ref[...]], axis=1)
        logits = jnp.dot(pooled, wfc_ref[...],
                         preferred_element_type=jnp.float32) + bfc_ref[...]
        m = jnp.max(logits, axis=1, keepdims=True)
        lse = m + jnp.log(jnp.sum(jnp.exp(logits - m), axis=1, keepdims=True))
        out_ref[...] = logits - lse


def kernel(x, edge_index, batch, Wc1, bc1, W11, b11, W12, b12, W13, b13,
           Wc2, bc2, W21, b21, W22, b22, W23, b23, Wfc2, bfc2):
    n, dx = x.shape
    e = edge_index.shape[1]
    epw = e // _NW
    c = 125
    nchunk = epw // c
    rlast = n - (_NS - 1) * ((n // _NS) // 8 * 8)

    src3 = edge_index[0].reshape(_NW, nchunk, c)
    dst3 = edge_index[1].reshape(_NW, nchunk, c)
    zeros64 = jnp.zeros((rlast, 64), jnp.float32)
    zeros80 = jnp.zeros((rlast, 80), jnp.float32)

    Wam1 = jnp.concatenate([W11, W12, W13], axis=1)
    bam1 = jnp.concatenate([b11, b12, b13])[None, :]
    Wam2 = jnp.concatenate([W21, W22, W23], axis=1)
    bam2 = jnp.concatenate([b21, b22, b23])[None, :]

    bsz = 1000
    nb = n // bsz
    full = lambda shape: pl.BlockSpec(shape, lambda i: tuple(0 for _ in shape))

    am1 = pl.pallas_call(
        _tc1_body,
        grid=(nb,),
        in_specs=[
            pl.BlockSpec((bsz, dx), lambda i: (i, 0)),
            full((dx, 96)),
            full((1, 96)),
        ],
        out_specs=pl.BlockSpec((bsz, 80), lambda i: (i, 0)),
        out_shape=jax.ShapeDtypeStruct((n, 80), jnp.float32),
    )(x, Wam1, bam1)

    segsum64 = _make_segsum(n, e, 64)
    xa = x[:, :64]
    xb = x[:, 64:dx]
    aggxa = segsum64(src3, dst3, xa, zeros64)
    aggxb = segsum64(src3, dst3, xb, zeros64)

    h1a, h1b = pl.pallas_call(
        _tc2_body,
        grid=(nb,),
        in_specs=[
            pl.BlockSpec((bsz, 80), lambda i: (i, 0)),
            pl.BlockSpec((_NC, bsz, 64), lambda i: (0, i, 0)),
            pl.BlockSpec((_NC, bsz, 64), lambda i: (0, i, 0)),
            full((dx, 64)),
            full((1, 64)),
        ],
        out_specs=[
            pl.BlockSpec((bsz, 64), lambda i: (i, 0)),
            pl.BlockSpec((bsz, 80), lambda i: (i, 0)),
        ],
        out_shape=[
            jax.ShapeDtypeStruct((n, 64), jnp.float32),
            jax.ShapeDtypeStruct((n, 80), jnp.float32),
        ],
    )(am1, aggxa, aggxb, Wc1, bc1[None, :])

    segsum80 = _make_segsum(n, e, 80)
    aggha = segsum64(src3, dst3, h1a, zeros64)
    agghb = segsum80(src3, dst3, h1b, zeros80)

    batch3 = batch.reshape(nb, 1, bsz)
    out = pl.pallas_call(
        functools.partial(_tc3_body, nb),
        grid=(nb,),
        in_specs=[
            pl.BlockSpec((bsz, 64), lambda i: (i, 0)),
            pl.BlockSpec((bsz, 80), lambda i: (i, 0)),
            pl.BlockSpec((_NC, bsz, 64), lambda i: (0, i, 0)),
            pl.BlockSpec((_NC, bsz, 80), lambda i: (0, i, 0)),
            full((144, 96)),
            full((1, 96)),
            full((144, 64)),
            full((1, 64)),
            pl.BlockSpec((1, 1, bsz), lambda i: (i, 0, 0)),
            full((288, 2)),
            full((1, 2)),
        ],
        out_specs=pl.BlockSpec((_G, 2), lambda i: (0, 0)),
        out_shape=jax.ShapeDtypeStruct((_G, 2), jnp.float32),
        scratch_shapes=[
            pltpu.VMEM((_G, 144), jnp.float32),
            pltpu.VMEM((_G, 144), jnp.float32),
            pltpu.VMEM((_G, 144), jnp.float32),
        ],
    )(h1a, h1b, aggha, agghb, Wam2, bam2, Wc2, bc2[None, :], batch3,
      Wfc2, bfc2[None, :])
    return out


# speedup vs baseline: 1.0624x; 1.0003x over previous
import functools

import jax
import jax.numpy as jnp
from jax import lax
from jax.experimental import pallas as pl
from jax.experimental.pallas import tpu as pltpu
from jax.experimental.pallas import tpu_sc as plsc

_NC = 2
_NS = 16
_NW = _NC * _NS
_G = 64


def _make_segsum(n, e, d):
    epw = e // _NW
    c = 125
    nchunk = epw // c
    r0 = (n // _NS) // 8 * 8
    rlast = n - (_NS - 1) * r0
    mesh = plsc.VectorSubcoreMesh(core_axis_name="c", subcore_axis_name="s")

    @functools.partial(
        pl.kernel,
        out_type=jax.ShapeDtypeStruct((_NC, n, d), jnp.float32),
        mesh=mesh,
        compiler_params=pltpu.CompilerParams(use_tc_tiling_on_sc=False),
        scratch_types=[
            pltpu.VMEM((nchunk, c), jnp.int32),
            pltpu.VMEM((nchunk, c), jnp.int32),
            [pltpu.VMEM((c, d), jnp.float32)] * 4,
            pltpu.VMEM_SHARED((n, d), jnp.float32),
            [pltpu.SemaphoreType.DMA] * 4,
            [pltpu.SemaphoreType.DMA] * 4,
            pltpu.SemaphoreType.DMA,
        ],
    )
    def segsum(src_hbm, dst_hbm, y_hbm, zeros_hbm, out_hbm,
               idx_s, idx_d, rows, agg_sh, sem_g, sem_s, sem_i):
        cid = lax.axis_index("c")
        sid = lax.axis_index("s")
        wid = sid * _NC + cid

        @pl.when(sid < _NS - 1)
        def _():
            pltpu.sync_copy(zeros_hbm.at[pl.ds(0, r0)],
                            agg_sh.at[pl.ds(sid * r0, r0)])

        @pl.when(sid == _NS - 1)
        def _():
            pltpu.sync_copy(zeros_hbm,
                            agg_sh.at[pl.ds((_NS - 1) * r0, rlast)])

        pltpu.async_copy(src_hbm.at[wid], idx_s, sem_i)
        pltpu.async_copy(dst_hbm.at[wid], idx_d, sem_i).wait()
        pltpu.make_async_copy(src_hbm.at[wid], idx_s, sem_i).wait()
        plsc.subcore_barrier()

        nbuf = 4
        for j in range(nbuf):
            pltpu.async_copy(y_hbm.at[idx_s.at[j]], rows[j], sem_g[j])

        def body(k, carry):
            base = nbuf * k
            for j in range(nbuf):
                ch = base + j
                pltpu.make_async_copy(
                    y_hbm.at[idx_s.at[ch]], rows[j], sem_g[j]).wait()
                pltpu.async_copy(
                    rows[j], agg_sh.at[idx_d.at[ch]], sem_s[j], add=True)
            for j in range(nbuf):
                ch = base + j
                pltpu.make_async_copy(
                    rows[j], agg_sh.at[idx_d.at[ch]], sem_s[j]).wait()

                def refill(jj=j, nch=base + nbuf + j):
                    pltpu.async_copy(
                        y_hbm.at[idx_s.at[nch]], rows[jj], sem_g[jj])

                pl.when(base + nbuf + j < nchunk)(refill)
            return carry

        lax.fori_loop(0, nchunk // nbuf, body, 0, unroll=False)
        plsc.subcore_barrier()

        @pl.when(sid < _NS - 1)
        def _():
            pltpu.sync_copy(agg_sh.at[pl.ds(sid * r0, r0)],
                            out_hbm.at[cid, pl.ds(sid * r0, r0)])

        @pl.when(sid == _NS - 1)
        def _():
            pltpu.sync_copy(agg_sh.at[pl.ds((_NS - 1) * r0, rlast)],
                            out_hbm.at[cid, pl.ds((_NS - 1) * r0, rlast)])

    return segsum


def _tc2_body(x_ref, wam_ref, bam_ref, agga_ref, aggb_ref, wc_ref, bc_ref,
              h1a_ref, h1b_ref):
    z = jnp.dot(x_ref[...], wam_ref[...], preferred_element_type=jnp.float32)
    z = z + bam_ref[...]
    a1 = jnp.maximum(z[:, :64], 0.0)
    m1 = jnp.maximum(z[:, 64:80], 0.0) * jnp.maximum(z[:, 80:96], 0.0)
    agg = jnp.concatenate(
        [agga_ref[0] + agga_ref[1], aggb_ref[0] + aggb_ref[1]], axis=1)
    c1 = jnp.maximum(
        jnp.dot(agg, wc_ref[...], preferred_element_type=jnp.float32)
        + bc_ref[...], 0.0)
    h1a_ref[...] = a1
    h1b_ref[...] = jnp.concatenate([c1, m1], axis=1)


def _tc3_body(nb, h1a_ref, h1b_ref, agga_ref, aggb_ref, w_ref, b_ref,
              wc_ref, bc_ref, batch_ref, wfc_ref, bfc_ref,
              out_ref, sum_ref, mx_ref, cnt_ref):
    i = pl.program_id(0)

    @pl.when(i == 0)
    def _():
        sum_ref[...] = jnp.zeros_like(sum_ref)
        mx_ref[...] = jnp.zeros_like(mx_ref)
        cnt_ref[...] = jnp.zeros_like(cnt_ref)

    agg = jnp.concatenate(
        [agga_ref[0] + agga_ref[1], aggb_ref[0] + aggb_ref[1]], axis=1)
    c2 = jnp.maximum(
        jnp.dot(agg, wc_ref[...], preferred_element_type=jnp.float32)
        + bc_ref[...], 0.0)
    h1 = jnp.concatenate([h1a_ref[...], h1b_ref[...]], axis=1)
    z = jnp.dot(h1, w_ref[...],
                preferred_element_type=jnp.float32) + b_ref[...]
    a2 = jnp.maximum(z[:, :64], 0.0)
    m2 = jnp.maximum(z[:, 64:80], 0.0) * jnp.maximum(z[:, 80:96], 0.0)
    h2 = jnp.concatenate([a2, c2, m2], axis=1)

    bsz = h2.shape[0]
    bid = batch_ref[0, 0, :].reshape(bsz, 1)
    g_lo = batch_ref[0, 0, 0]
    g_hi = batch_ref[0, 0, bsz - 1]

    def seg(g, carry):
        msk = (bid == g).astype(jnp.float32)
        mh = h2 * msk
        s = jnp.sum(mh, axis=0, keepdims=True)
        mx = jnp.max(mh, axis=0, keepdims=True)
        cnt = jnp.sum(msk)
        sum_ref[pl.ds(g, 1), :] += s
        mx_ref[pl.ds(g, 1), :] = jnp.maximum(mx_ref[pl.ds(g, 1), :], mx)
        cnt_ref[pl.ds(g, 1), :] += cnt
        return carry

    lax.fori_loop(g_lo, g_hi + 1, seg, 0)

    @pl.when(i == nb - 1)
    def _():
        mean = sum_ref[...] / jnp.maximum(cnt_ref[...], 1.0)
        pooled = jnp.concatenate([mean, mx_ref[...]], axis=1)
        logits = jnp.dot(pooled, wfc_ref[...],
                         preferred_element_type=jnp.float32) + bfc_ref[...]
        m = jnp.max(logits, axis=1, keepdims=True)
        lse = m + jnp.log(jnp.sum(jnp.exp(logits - m), axis=1, keepdims=True))
        out_ref[...] = logits - lse


def kernel(x, edge_index, batch, Wc1, bc1, W11, b11, W12, b12, W13, b13,
           Wc2, bc2, W21, b21, W22, b22, W23, b23, Wfc2, bfc2):
    n, dx = x.shape
    e = edge_index.shape[1]
    epw = e // _NW
    c = 125
    nchunk = epw // c
    rlast = n - (_NS - 1) * ((n // _NS) // 8 * 8)

    src3 = edge_index[0].reshape(_NW, nchunk, c)
    dst3 = edge_index[1].reshape(_NW, nchunk, c)
    zeros64 = jnp.zeros((rlast, 64), jnp.float32)
    zeros80 = jnp.zeros((rlast, 80), jnp.float32)

    Wam1 = jnp.concatenate([W11, W12, W13], axis=1)
    bam1 = jnp.concatenate([b11, b12, b13])[None, :]
    Wam2 = jnp.concatenate([W21, W22, W23], axis=1)
    bam2 = jnp.concatenate([b21, b22, b23])[None, :]

    bsz = 1000
    nb = n // bsz
    full = lambda shape: pl.BlockSpec(shape, lambda i: tuple(0 for _ in shape))

    segsum64 = _make_segsum(n, e, 64)
    xa = x[:, :64]
    xb = x[:, 64:dx]
    aggxa = segsum64(src3, dst3, xa, zeros64)
    aggxb = segsum64(src3, dst3, xb, zeros64)

    h1a, h1b = pl.pallas_call(
        _tc2_body,
        grid=(nb,),
        in_specs=[
            pl.BlockSpec((bsz, dx), lambda i: (i, 0)),
            full((dx, 96)),
            full((1, 96)),
            pl.BlockSpec((_NC, bsz, 64), lambda i: (0, i, 0)),
            pl.BlockSpec((_NC, bsz, 64), lambda i: (0, i, 0)),
            full((dx, 64)),
            full((1, 64)),
        ],
        out_specs=[
            pl.BlockSpec((bsz, 64), lambda i: (i, 0)),
            pl.BlockSpec((bsz, 80), lambda i: (i, 0)),
        ],
        out_shape=[
            jax.ShapeDtypeStruct((n, 64), jnp.float32),
            jax.ShapeDtypeStruct((n, 80), jnp.float32),
        ],
    )(x, Wam1, bam1, aggxa, aggxb, Wc1, bc1[None, :])

    segsum80 = _make_segsum(n, e, 80)
    aggha = segsum64(src3, dst3, h1a, zeros64)
    agghb = segsum80(src3, dst3, h1b, zeros80)

    batch3 = batch.reshape(nb, 1, bsz)
    out = pl.pallas_call(
        functools.partial(_tc3_body, nb),
        grid=(nb,),
        in_specs=[
            pl.BlockSpec((bsz, 64), lambda i: (i, 0)),
            pl.BlockSpec((bsz, 80), lambda i: (i, 0)),
            pl.BlockSpec((_NC, bsz, 64), lambda i: (0, i, 0)),
            pl.BlockSpec((_NC, bsz, 80), lambda i: (0, i, 0)),
            full((144, 96)),
            full((1, 96)),
            full((144, 64)),
            full((1, 64)),
            pl.BlockSpec((1, 1, bsz), lambda i: (i, 0, 0)),
            full((288, 2)),
            full((1, 2)),
        ],
        out_specs=pl.BlockSpec((_G, 2), lambda i: (0, 0)),
        out_shape=jax.ShapeDtypeStruct((_G, 2), jnp.float32),
        scratch_shapes=[
            pltpu.VMEM((_G, 144), jnp.float32),
            pltpu.VMEM((_G, 144), jnp.float32),
            pltpu.VMEM((_G, 144), jnp.float32),
        ],
    )(h1a, h1b, aggha, agghb, Wam2, bam2, Wc2, bc2[None, :], batch3,
      Wfc2, bfc2[None, :])
    return out
